# baseline (speedup 1.0000x reference)
import functools

import jax
import jax.numpy as jnp
from jax import lax
from jax.experimental import pallas as pl
from jax.experimental.pallas import tpu as pltpu
from jax.experimental.pallas import tpu_sc as plsc

N = 10000
E = 160000
NP = 10240
NDUMP = N
NC = 2
NS = 16
EPAD = 163840
NBATCH = EPAD // (NS * 128)
ROWS_PER_TILE = NP // NS
BN = 1024
GRID = NP // BN

_mesh = plsc.VectorSubcoreMesh(core_axis_name="c", subcore_axis_name="s")


@functools.partial(
    pl.kernel,
    out_type=jax.ShapeDtypeStruct((NC, NP), jnp.float32),
    mesh=_mesh,
    scratch_types=[
        pltpu.VMEM((NBATCH, 128), jnp.int32),
        pltpu.VMEM((128,), jnp.float32),
        pltpu.VMEM_SHARED((NP,), jnp.float32),
    ],
)
def _deg_kernel(idx_hbm, zeros_hbm, ones_hbm, out_hbm, idx_v, ones_v, deg_sp):
    c = lax.axis_index("c")
    s = lax.axis_index("s")
    pltpu.sync_copy(ones_hbm, ones_v)
    pltpu.sync_copy(zeros_hbm.at[pl.ds(0, ROWS_PER_TILE)],
                    deg_sp.at[pl.ds(s * ROWS_PER_TILE, ROWS_PER_TILE)])
    pltpu.sync_copy(idx_hbm.at[c, s], idx_v)
    plsc.subcore_barrier()

    def body(b, carry):
        pltpu.sync_copy(ones_v, deg_sp.at[idx_v.at[b]], add=True)
        return carry

    lax.fori_loop(0, NBATCH, body, None)
    plsc.subcore_barrier()

    @pl.when(s == 0)
    def _():
        pltpu.sync_copy(deg_sp, out_hbm.at[c])


HB = NBATCH // 2
GS = 8
QS = 128 // GS


def _make_agg_kernel(R):
    scratch = [
        pltpu.VMEM((HB, 128), jnp.int32),
        pltpu.VMEM((HB, 128), jnp.int32),
        pltpu.VMEM((2, 128, 128), jnp.float32),
        pltpu.VMEM_SHARED((NP, 128), jnp.float32),
        pltpu.SemaphoreType.DMA,
        pltpu.SemaphoreType.DMA,
    ]

    @functools.partial(
        pl.kernel,
        out_type=jax.ShapeDtypeStruct((R, NC, NP, 128), jnp.float32),
        mesh=_mesh,
        scratch_types=scratch,
    )
    def agg(*refs):
        tables = refs[:R]
        src_hbm, dst_hbm, zeros_hbm, out_hbm = refs[R:R + 4]
        src_v, dst_v, rows_v, agg_sp, gsem, ssem = refs[R + 4:]
        c = lax.axis_index("c")
        s = lax.axis_index("s")
        for r in range(R):
            pltpu.sync_copy(
                zeros_hbm,
                agg_sp.at[pl.ds(s * ROWS_PER_TILE, ROWS_PER_TILE)])
            plsc.subcore_barrier()
            table = tables[r]
            for h in range(NBATCH // HB):
                pltpu.sync_copy(src_hbm.at[c, s, pl.ds(h * HB, HB)], src_v)
                pltpu.sync_copy(dst_hbm.at[s, pl.ds(h * HB, HB)], dst_v)
                for q in range(GS):
                    pltpu.async_copy(
                        table.at[src_v.at[0, pl.ds(q * QS, QS)]],
                        rows_v.at[0, pl.ds(q * QS, QS)], gsem)

                def body(b, carry, table=table):
                    j = lax.rem(b, 2)
                    jn = lax.rem(b + 1, 2)

                    @pl.when(b > 0)
                    def _():
                        pltpu.make_async_copy(
                            rows_v.at[jn],
                            agg_sp.at[dst_v.at[b - 1]], ssem).wait()

                    @pl.when(b + 1 < HB)
                    def _():
                        for q in range(GS):
                            pltpu.async_copy(
                                table.at[src_v.at[b + 1, pl.ds(q * QS, QS)]],
                                rows_v.at[jn, pl.ds(q * QS, QS)], gsem)

                    for q in range(GS):
                        pltpu.make_async_copy(
                            table.at[src_v.at[b, pl.ds(q * QS, QS)]],
                            rows_v.at[j, pl.ds(q * QS, QS)], gsem).wait()

                    pltpu.async_copy(
                        rows_v.at[j], agg_sp.at[dst_v.at[b]], ssem, add=True)
                    return carry

                lax.fori_loop(0, HB, body, None)
                pltpu.make_async_copy(
                    rows_v.at[(HB - 1) % 2],
                    agg_sp.at[dst_v.at[HB - 1]], ssem).wait()
            plsc.subcore_barrier()
            pltpu.sync_copy(
                agg_sp.at[pl.ds(s * ROWS_PER_TILE, ROWS_PER_TILE)],
                out_hbm.at[r, c, pl.ds(s * ROWS_PER_TILE, ROWS_PER_TILE)])

    return agg


_agg1 = _make_agg_kernel(1)
_agg2 = _make_agg_kernel(2)


def _norm(deg_blk):
    return lax.rsqrt(jnp.maximum(deg_blk, 1.0))


def _pre_body(x_ref, dout_ref, out_ref):
    xs = x_ref[...] * _norm(dout_ref[...])
    out_ref[0] = xs[:, :128]
    out_ref[1] = xs[:, 128:]


def _mm1_body(agg_ref, din_ref, dout_ref, w_ref, b_ref, out_ref):
    a = jnp.concatenate([agg_ref[0], agg_ref[1]], axis=1) * _norm(din_ref[...])
    h = jnp.dot(a, w_ref[...], preferred_element_type=jnp.float32) + b_ref[...]
    h = jnp.maximum(h, 0.0) * _norm(dout_ref[...])
    for j in range(4):
        out_ref[j] = h[:, j * 128:(j + 1) * 128]


def _mm2_body(agg_ref, din_ref, wc2_ref, bc2_ref, wl1_ref, bl1_ref, wo_ref,
              bo_ref, out_ref):
    a = jnp.concatenate([agg_ref[j] for j in range(4)], axis=1)
    a = a * _norm(din_ref[...])
    h = jnp.dot(a, wc2_ref[...], preferred_element_type=jnp.float32)
    h = jnp.maximum(h + bc2_ref[...], 0.0)
    h = jnp.dot(h, wl1_ref[...], preferred_element_type=jnp.float32)
    h = jnp.maximum(h + bl1_ref[...], 0.0)
    out_ref[...] = (jnp.dot(h, wo_ref[...], preferred_element_type=jnp.float32)
                    + bo_ref[...])


def _full(shape):
    return pl.BlockSpec(shape, lambda i: tuple(0 for _ in shape))


_pre_call = pl.pallas_call(
    _pre_body,
    grid=(GRID,),
    in_specs=[
        pl.BlockSpec((BN, 256), lambda i: (i, 0)),
        pl.BlockSpec((BN, 1), lambda i: (i, 0)),
    ],
    out_specs=pl.BlockSpec((2, BN, 128), lambda i: (0, i, 0)),
    out_shape=jax.ShapeDtypeStruct((2, NP, 128), jnp.float32),
)

_mm1_call = pl.pallas_call(
    _mm1_body,
    grid=(GRID,),
    in_specs=[
        pl.BlockSpec((2, BN, 128), lambda i: (0, i, 0)),
        pl.BlockSpec((BN, 1), lambda i: (i, 0)),
        pl.BlockSpec((BN, 1), lambda i: (i, 0)),
        _full((256, 512)),
        _full((1, 512)),
    ],
    out_specs=pl.BlockSpec((4, BN, 128), lambda i: (0, i, 0)),
    out_shape=jax.ShapeDtypeStruct((4, NP, 128), jnp.float32),
)

_mm2_call = pl.pallas_call(
    _mm2_body,
    grid=(GRID,),
    in_specs=[
        pl.BlockSpec((4, BN, 128), lambda i: (0, i, 0)),
        pl.BlockSpec((BN, 1), lambda i: (i, 0)),
        _full((512, 512)),
        _full((1, 512)),
        _full((512, 512)),
        _full((1, 512)),
        _full((512, 128)),
        _full((1, 128)),
    ],
    out_specs=pl.BlockSpec((BN, 128), lambda i: (i, 0)),
    out_shape=jax.ShapeDtypeStruct((NP, 128), jnp.float32),
)


def kernel(x, edge_index, Wc1, bc1, Wc2, bc2, Wl1, bl1, Wo, bo):
    src = edge_index[0].astype(jnp.int32)
    dst = edge_index[1].astype(jnp.int32)
    pad = EPAD - E
    src_g = jnp.concatenate([src, jnp.zeros((pad,), jnp.int32)])
    dst_p = jnp.concatenate([dst, jnp.full((pad,), NDUMP, jnp.int32)])
    src_d = jnp.concatenate([src, jnp.full((pad,), NDUMP, jnp.int32)])

    src_idx = jnp.stack([src_g, src_g + NP]).reshape(NC, NS, NBATCH, 128)
    dst_idx = dst_p.reshape(NS, NBATCH, 128)
    deg_idx = jnp.stack([src_d, dst_p]).reshape(NC, NS, NBATCH, 128)

    zeros_flat = jnp.zeros((ROWS_PER_TILE,), jnp.float32)
    zeros = jnp.zeros((ROWS_PER_TILE, 128), jnp.float32)
    ones = jnp.ones((128,), jnp.float32)

    degs = _deg_kernel(deg_idx, zeros_flat, ones)
    deg_out = degs[0].reshape(NP, 1)
    deg_in = degs[1].reshape(NP, 1)

    x_pad = jnp.pad(x, ((0, NP - N), (0, 0)))

    xs_parts = _pre_call(x_pad, deg_out)
    table1 = xs_parts.reshape(2 * NP, 128)
    agg1 = _agg1(table1, src_idx, dst_idx, zeros)
    h1s_parts = _mm1_call(agg1.reshape(NC, NP, 128), deg_in, deg_out,
                          Wc1, bc1.reshape(1, 512))

    tables2 = h1s_parts.reshape(2, 2 * NP, 128)
    agg2 = _agg2(tables2[0], tables2[1], src_idx, dst_idx, zeros)
    out = _mm2_call(agg2.reshape(4, NP, 128), deg_in,
                    Wc2, bc2.reshape(1, 512),
                    Wl1, bl1.reshape(1, 512),
                    Wo, bo.reshape(1, 128))
    return out[:N]


# speedup vs baseline: 1.0005x; 1.0005x over previous
import functools

import jax
import jax.numpy as jnp
from jax import lax
from jax.experimental import pallas as pl
from jax.experimental.pallas import tpu as pltpu
from jax.experimental.pallas import tpu_sc as plsc

N = 10000
E = 160000
NP = 10240
NDUMP = N
NC = 2
NS = 16
EPAD = 163840
NBATCH = EPAD // (NS * 128)
ROWS_PER_TILE = NP // NS
BN = 1024
GRID = NP // BN

_mesh = plsc.VectorSubcoreMesh(core_axis_name="c", subcore_axis_name="s")


@functools.partial(
    pl.kernel,
    out_type=jax.ShapeDtypeStruct((NC, NP), jnp.float32),
    mesh=_mesh,
    scratch_types=[
        pltpu.VMEM((NBATCH, 128), jnp.int32),
        pltpu.VMEM((128,), jnp.float32),
        pltpu.VMEM_SHARED((NP,), jnp.float32),
    ],
)
def _deg_kernel(idx_hbm, zeros_hbm, ones_hbm, out_hbm, idx_v, ones_v, deg_sp):
    c = lax.axis_index("c")
    s = lax.axis_index("s")
    pltpu.sync_copy(ones_hbm, ones_v)
    pltpu.sync_copy(zeros_hbm.at[pl.ds(0, ROWS_PER_TILE)],
                    deg_sp.at[pl.ds(s * ROWS_PER_TILE, ROWS_PER_TILE)])
    pltpu.sync_copy(idx_hbm.at[c, s], idx_v)
    plsc.subcore_barrier()

    def body(b, carry):
        pltpu.sync_copy(ones_v, deg_sp.at[idx_v.at[b]], add=True)
        return carry

    lax.fori_loop(0, NBATCH, body, None)
    plsc.subcore_barrier()

    @pl.when(s == 0)
    def _():
        pltpu.sync_copy(deg_sp, out_hbm.at[c])


HB = NBATCH // 2
GS = 4
QS = 128 // GS


def _make_agg_kernel(R):
    scratch = [
        pltpu.VMEM((HB, 128), jnp.int32),
        pltpu.VMEM((HB, 128), jnp.int32),
        pltpu.VMEM((2, 128, 128), jnp.float32),
        pltpu.VMEM_SHARED((NP, 128), jnp.float32),
        pltpu.SemaphoreType.DMA,
        pltpu.SemaphoreType.DMA,
    ]

    @functools.partial(
        pl.kernel,
        out_type=jax.ShapeDtypeStruct((R, NC, NP, 128), jnp.float32),
        mesh=_mesh,
        scratch_types=scratch,
    )
    def agg(*refs):
        tables = refs[:R]
        src_hbm, dst_hbm, zeros_hbm, out_hbm = refs[R:R + 4]
        src_v, dst_v, rows_v, agg_sp, gsem, ssem = refs[R + 4:]
        c = lax.axis_index("c")
        s = lax.axis_index("s")
        for r in range(R):
            pltpu.sync_copy(
                zeros_hbm,
                agg_sp.at[pl.ds(s * ROWS_PER_TILE, ROWS_PER_TILE)])
            plsc.subcore_barrier()
            table = tables[r]
            for h in range(NBATCH // HB):
                pltpu.sync_copy(src_hbm.at[c, s, pl.ds(h * HB, HB)], src_v)
                pltpu.sync_copy(dst_hbm.at[s, pl.ds(h * HB, HB)], dst_v)
                for q in range(GS):
                    pltpu.async_copy(
                        table.at[src_v.at[0, pl.ds(q * QS, QS)]],
                        rows_v.at[0, pl.ds(q * QS, QS)], gsem)

                def body(b, carry, table=table):
                    j = lax.rem(b, 2)
                    jn = lax.rem(b + 1, 2)

                    @pl.when(b > 0)
                    def _():
                        pltpu.make_async_copy(
                            rows_v.at[jn],
                            agg_sp.at[dst_v.at[b - 1]], ssem).wait()

                    @pl.when(b + 1 < HB)
                    def _():
                        for q in range(GS):
                            pltpu.async_copy(
                                table.at[src_v.at[b + 1, pl.ds(q * QS, QS)]],
                                rows_v.at[jn, pl.ds(q * QS, QS)], gsem)

                    pltpu.make_async_copy(
                        table.at[src_v.at[b]], rows_v.at[j], gsem).wait()

                    pltpu.async_copy(
                        rows_v.at[j], agg_sp.at[dst_v.at[b]], ssem, add=True)
                    return carry

                lax.fori_loop(0, HB, body, None)
                pltpu.make_async_copy(
                    rows_v.at[(HB - 1) % 2],
                    agg_sp.at[dst_v.at[HB - 1]], ssem).wait()
            plsc.subcore_barrier()
            pltpu.sync_copy(
                agg_sp.at[pl.ds(s * ROWS_PER_TILE, ROWS_PER_TILE)],
                out_hbm.at[r, c, pl.ds(s * ROWS_PER_TILE, ROWS_PER_TILE)])

    return agg


_agg1 = _make_agg_kernel(1)
_agg2 = _make_agg_kernel(2)


def _norm(deg_blk):
    return lax.rsqrt(jnp.maximum(deg_blk, 1.0))


def _pre_body(x_ref, dout_ref, out_ref):
    xs = x_ref[...] * _norm(dout_ref[...])
    out_ref[0] = xs[:, :128]
    out_ref[1] = xs[:, 128:]


def _mm1_body(agg_ref, din_ref, dout_ref, w_ref, b_ref, out_ref):
    a = jnp.concatenate([agg_ref[0], agg_ref[1]], axis=1) * _norm(din_ref[...])
    h = jnp.dot(a, w_ref[...], preferred_element_type=jnp.float32) + b_ref[...]
    h = jnp.maximum(h, 0.0) * _norm(dout_ref[...])
    for j in range(4):
        out_ref[j] = h[:, j * 128:(j + 1) * 128]


def _mm2_body(agg_ref, din_ref, wc2_ref, bc2_ref, wl1_ref, bl1_ref, wo_ref,
              bo_ref, out_ref):
    a = jnp.concatenate([agg_ref[j] for j in range(4)], axis=1)
    a = a * _norm(din_ref[...])
    h = jnp.dot(a, wc2_ref[...], preferred_element_type=jnp.float32)
    h = jnp.maximum(h + bc2_ref[...], 0.0)
    h = jnp.dot(h, wl1_ref[...], preferred_element_type=jnp.float32)
    h = jnp.maximum(h + bl1_ref[...], 0.0)
    out_ref[...] = (jnp.dot(h, wo_ref[...], preferred_element_type=jnp.float32)
                    + bo_ref[...])


def _full(shape):
    return pl.BlockSpec(shape, lambda i: tuple(0 for _ in shape))


_pre_call = pl.pallas_call(
    _pre_body,
    grid=(GRID,),
    in_specs=[
        pl.BlockSpec((BN, 256), lambda i: (i, 0)),
        pl.BlockSpec((BN, 1), lambda i: (i, 0)),
    ],
    out_specs=pl.BlockSpec((2, BN, 128), lambda i: (0, i, 0)),
    out_shape=jax.ShapeDtypeStruct((2, NP, 128), jnp.float32),
)

_mm1_call = pl.pallas_call(
    _mm1_body,
    grid=(GRID,),
    in_specs=[
        pl.BlockSpec((2, BN, 128), lambda i: (0, i, 0)),
        pl.BlockSpec((BN, 1), lambda i: (i, 0)),
        pl.BlockSpec((BN, 1), lambda i: (i, 0)),
        _full((256, 512)),
        _full((1, 512)),
    ],
    out_specs=pl.BlockSpec((4, BN, 128), lambda i: (0, i, 0)),
    out_shape=jax.ShapeDtypeStruct((4, NP, 128), jnp.float32),
)

_mm2_call = pl.pallas_call(
    _mm2_body,
    grid=(GRID,),
    in_specs=[
        pl.BlockSpec((4, BN, 128), lambda i: (0, i, 0)),
        pl.BlockSpec((BN, 1), lambda i: (i, 0)),
        _full((512, 512)),
        _full((1, 512)),
        _full((512, 512)),
        _full((1, 512)),
        _full((512, 128)),
        _full((1, 128)),
    ],
    out_specs=pl.BlockSpec((BN, 128), lambda i: (i, 0)),
    out_shape=jax.ShapeDtypeStruct((NP, 128), jnp.float32),
)


def kernel(x, edge_index, Wc1, bc1, Wc2, bc2, Wl1, bl1, Wo, bo):
    src = edge_index[0].astype(jnp.int32)
    dst = edge_index[1].astype(jnp.int32)
    pad = EPAD - E
    src_g = jnp.concatenate([src, jnp.zeros((pad,), jnp.int32)])
    dst_p = jnp.concatenate([dst, jnp.full((pad,), NDUMP, jnp.int32)])
    src_d = jnp.concatenate([src, jnp.full((pad,), NDUMP, jnp.int32)])

    src_idx = jnp.stack([src_g, src_g + NP]).reshape(NC, NS, NBATCH, 128)
    dst_idx = dst_p.reshape(NS, NBATCH, 128)
    deg_idx = jnp.stack([src_d, dst_p]).reshape(NC, NS, NBATCH, 128)

    zeros_flat = jnp.zeros((ROWS_PER_TILE,), jnp.float32)
    zeros = jnp.zeros((ROWS_PER_TILE, 128), jnp.float32)
    ones = jnp.ones((128,), jnp.float32)

    degs = _deg_kernel(deg_idx, zeros_flat, ones)
    deg_out = degs[0].reshape(NP, 1)
    deg_in = degs[1].reshape(NP, 1)

    x_pad = jnp.pad(x, ((0, NP - N), (0, 0)))

    xs_parts = _pre_call(x_pad, deg_out)
    table1 = xs_parts.reshape(2 * NP, 128)
    agg1 = _agg1(table1, src_idx, dst_idx, zeros)
    h1s_parts = _mm1_call(agg1.reshape(NC, NP, 128), deg_in, deg_out,
                          Wc1, bc1.reshape(1, 512))

    tables2 = h1s_parts.reshape(2, 2 * NP, 128)
    agg2 = _agg2(tables2[0], tables2[1], src_idx, dst_idx, zeros)
    out = _mm2_call(agg2.reshape(4, NP, 128), deg_in,
                    Wc2, bc2.reshape(1, 512),
                    Wl1, bl1.reshape(1, 512),
                    Wo, bo.reshape(1, 128))
    return out[:N]
